# baseline (speedup 1.0000x reference)
import functools
import jax
import jax.numpy as jnp
from jax import lax
from jax.experimental import pallas as pl
from jax.experimental.pallas import tpu as pltpu
from jax.experimental.pallas import tpu_sc as plsc

N = 8192
D = 4096
OUT = 4096
NE = 28
NEP = 32
R = 8
RMOE = NE * R
SCALING = 2.0

BN = 256
NW = 32
TPW = N // NW
NEG = -3.0e38


def _logits_kernel(x_ref, wgp_ref, lg_ref):
    xb = x_ref[...].astype(jnp.bfloat16)
    lg = jax.lax.dot_general(wgp_ref[...], xb, (((1,), (1,)), ((), ())),
                             preferred_element_type=jnp.float32)
    ii = jax.lax.broadcasted_iota(jnp.int32, (NEP, BN), 0)
    lg_ref[...] = jnp.where(ii < NE, lg, NEG)


def _sc_router(lg_hbm, g_hbm, lg_v, g_v, sem):
    wid = lax.axis_index("s") * 2 + lax.axis_index("c")
    base = wid * TPW
    pltpu.sync_copy(lg_hbm.at[:, pl.ds(base, TPW)], lg_v)

    def group(gi, _):
        off = gi * 16
        m1 = jnp.full((16,), NEG, jnp.float32)
        m2 = jnp.full((16,), NEG, jnp.float32)
        i1 = jnp.zeros((16,), jnp.int32)
        i2 = jnp.zeros((16,), jnp.int32)

        def scan_e(e, carry):
            m1, i1, m2, i2 = carry
            v = lg_v[e, pl.ds(off, 16)]
            gt1 = v > m1
            gt2 = v > m2
            m2n = jnp.where(gt1, m1, jnp.where(gt2, v, m2))
            i2n = jnp.where(gt1, i1, jnp.where(gt2, e, i2))
            m1n = jnp.where(gt1, v, m1)
            i1n = jnp.where(gt1, e, i1)
            return (m1n, i1n, m2n, i2n)

        m1, i1, m2, i2 = lax.fori_loop(0, NE, scan_e, (m1, i1, m2, i2))
        ex = jnp.exp(m2 - m1)
        w1 = 1.0 / (1.0 + ex)
        w2 = 1.0 - w1
        g_v[0, pl.ds(off, 16)] = w1
        g_v[1, pl.ds(off, 16)] = w2
        g_v[2, pl.ds(off, 16)] = i1.astype(jnp.float32)
        g_v[3, pl.ds(off, 16)] = i2.astype(jnp.float32)
        g_v[4, pl.ds(off, 16)] = w1
        g_v[5, pl.ds(off, 16)] = w2
        g_v[6, pl.ds(off, 16)] = w1
        g_v[7, pl.ds(off, 16)] = w2
        return 0

    lax.fori_loop(0, TPW // 16, group, 0)
    pltpu.sync_copy(g_v, g_hbm.at[:, pl.ds(base, TPW)])


def _main_kernel(x_ref, g_ref, wa_ref, wb_ref, wbt_ref, out_ref):
    xb = x_ref[...].astype(jnp.bfloat16)
    w1 = g_ref[0:1, :].reshape(BN, 1)
    w2 = g_ref[1:2, :].reshape(BN, 1)
    i1 = g_ref[2:3, :].reshape(BN, 1).astype(jnp.int32)
    i2 = g_ref[3:4, :].reshape(BN, 1).astype(jnp.int32)
    colmap = jax.lax.broadcasted_iota(jnp.int32, (BN, RMOE), 1) // R
    gex = jnp.where(colmap == i1, w1, 0.0) + jnp.where(colmap == i2, w2, 0.0)
    h = jnp.dot(xb, wa_ref[...], preferred_element_type=jnp.float32)
    hw = (h * gex).astype(jnp.bfloat16)
    acc = jnp.dot(xb, wb_ref[...], preferred_element_type=jnp.float32)
    acc = acc + jnp.dot(hw, wbt_ref[...], preferred_element_type=jnp.float32)
    out_ref[...] = acc


@jax.jit
def kernel(x, W_base, W_gate, W_A, W_B):
    wgp = jnp.zeros((NEP, D), jnp.bfloat16).at[:NE].set(
        W_gate.astype(jnp.bfloat16))
    wa = W_A.T.astype(jnp.bfloat16)
    wb = W_base.T.astype(jnp.bfloat16)
    wbt = (SCALING * W_B).T.astype(jnp.bfloat16)

    lgT = pl.pallas_call(
        _logits_kernel,
        grid=(N // BN,),
        in_specs=[
            pl.BlockSpec((BN, D), lambda i: (i, 0)),
            pl.BlockSpec((NEP, D), lambda i: (0, 0)),
        ],
        out_specs=pl.BlockSpec((NEP, BN), lambda i: (0, i)),
        out_shape=jax.ShapeDtypeStruct((NEP, N), jnp.float32),
    )(x, wgp)

    g = pl.kernel(
        _sc_router,
        mesh=plsc.VectorSubcoreMesh(core_axis_name="c", subcore_axis_name="s"),
        out_type=jax.ShapeDtypeStruct((8, N), jnp.float32),
        scratch_types=[
            pltpu.VMEM((NEP, TPW), jnp.float32),
            pltpu.VMEM((8, TPW), jnp.float32),
            pltpu.SemaphoreType.DMA,
        ],
    )(lgT)

    return pl.pallas_call(
        _main_kernel,
        grid=(N // BN,),
        in_specs=[
            pl.BlockSpec((BN, D), lambda i: (i, 0)),
            pl.BlockSpec((8, BN), lambda i: (0, i)),
            pl.BlockSpec((D, RMOE), lambda i: (0, 0)),
            pl.BlockSpec((D, OUT), lambda i: (0, 0)),
            pl.BlockSpec((RMOE, OUT), lambda i: (0, 0)),
        ],
        out_specs=pl.BlockSpec((BN, OUT), lambda i: (i, 0)),
        out_shape=jax.ShapeDtypeStruct((N, OUT), jnp.float32),
        compiler_params=pltpu.CompilerParams(
            vmem_limit_bytes=100 * 1024 * 1024,
        ),
    )(x, g, wa, wb, wbt)


# speedup vs baseline: 1.1598x; 1.1598x over previous
import jax
import jax.numpy as jnp
from jax.experimental import pallas as pl
from jax.experimental.pallas import tpu as pltpu
from functools import partial

N = 8192
D = 4096
OUT = 4096
NE = 28
R = 8
RMOE = NE * R
SCALING = 2.0

BN = 256
WCH = OUT // 8


def _prep_kernel(wbase_ref, wgate_ref, wa_in_ref, wbmat_ref,
                 wb_ref, wg_ref, wa_ref, wbt_ref):
    i = pl.program_id(0)
    wb_ref[...] = wbase_ref[...].astype(jnp.bfloat16).T

    @pl.when(i == 0)
    def _():
        wg_ref[...] = wgate_ref[...].astype(jnp.bfloat16).T
        wa_ref[...] = wa_in_ref[...].astype(jnp.bfloat16).T
        wbt_ref[...] = (SCALING * wbmat_ref[...]).astype(jnp.bfloat16).T


def _fused_kernel(x_ref, wg_ref, wa_ref, wb_ref, wbt_ref, out_ref):
    xb = x_ref[...].astype(jnp.bfloat16)
    logits = jnp.dot(xb, wg_ref[...], preferred_element_type=jnp.float32)
    ii = jax.lax.broadcasted_iota(jnp.int32, (BN, NE), 1)
    m1 = jnp.max(logits, axis=-1, keepdims=True)
    i1 = jnp.min(jnp.where(logits == m1, ii, NE), axis=-1, keepdims=True)
    l2 = jnp.where(ii == i1, -jnp.inf, logits)
    m2 = jnp.max(l2, axis=-1, keepdims=True)
    i2 = jnp.min(jnp.where(l2 == m2, ii, NE), axis=-1, keepdims=True)
    e = jnp.exp(m2 - m1)
    w1 = 1.0 / (1.0 + e)
    w2 = 1.0 - w1
    colmap = jax.lax.broadcasted_iota(jnp.int32, (BN, RMOE), 1) // R
    gex = jnp.where(colmap == i1, w1, 0.0) + jnp.where(colmap == i2, w2, 0.0)
    h = jnp.dot(xb, wa_ref[...], preferred_element_type=jnp.float32)
    hw = (h * gex).astype(jnp.bfloat16)
    acc = jnp.dot(xb, wb_ref[...], preferred_element_type=jnp.float32)
    acc = acc + jnp.dot(hw, wbt_ref[...], preferred_element_type=jnp.float32)
    out_ref[...] = acc


@jax.jit
def kernel(x, W_base, W_gate, W_A, W_B):
    wb, wg, wa, wbt = pl.pallas_call(
        _prep_kernel,
        grid=(8,),
        in_specs=[
            pl.BlockSpec((WCH, D), lambda i: (i, 0)),
            pl.BlockSpec((NE, D), lambda i: (0, 0)),
            pl.BlockSpec((RMOE, D), lambda i: (0, 0)),
            pl.BlockSpec((OUT, RMOE), lambda i: (0, 0)),
        ],
        out_specs=[
            pl.BlockSpec((D, WCH), lambda i: (0, i)),
            pl.BlockSpec((D, NE), lambda i: (0, 0)),
            pl.BlockSpec((D, RMOE), lambda i: (0, 0)),
            pl.BlockSpec((RMOE, OUT), lambda i: (0, 0)),
        ],
        out_shape=[
            jax.ShapeDtypeStruct((D, OUT), jnp.bfloat16),
            jax.ShapeDtypeStruct((D, NE), jnp.bfloat16),
            jax.ShapeDtypeStruct((D, RMOE), jnp.bfloat16),
            jax.ShapeDtypeStruct((RMOE, OUT), jnp.bfloat16),
        ],
        compiler_params=pltpu.CompilerParams(
            vmem_limit_bytes=100 * 1024 * 1024,
        ),
    )(W_base, W_gate, W_A, W_B)

    return pl.pallas_call(
        _fused_kernel,
        grid=(N // BN,),
        in_specs=[
            pl.BlockSpec((BN, D), lambda i: (i, 0)),
            pl.BlockSpec((D, NE), lambda i: (0, 0)),
            pl.BlockSpec((D, RMOE), lambda i: (0, 0)),
            pl.BlockSpec((D, OUT), lambda i: (0, 0)),
            pl.BlockSpec((RMOE, OUT), lambda i: (0, 0)),
        ],
        out_specs=pl.BlockSpec((BN, OUT), lambda i: (i, 0)),
        out_shape=jax.ShapeDtypeStruct((N, OUT), jnp.float32),
        compiler_params=pltpu.CompilerParams(
            vmem_limit_bytes=100 * 1024 * 1024,
        ),
    )(x, wg, wa, wb, wbt)


# speedup vs baseline: 1.1680x; 1.0071x over previous
import jax
import jax.numpy as jnp
from jax.experimental import pallas as pl
from jax.experimental.pallas import tpu as pltpu
from functools import partial

N = 8192
D = 4096
OUT = 4096
NE = 28
R = 8
RMOE = NE * R
SCALING = 2.0

BN = 256
WCH = OUT // 8


def _prep_kernel(wbase_ref, wgate_ref, wa_in_ref, wbmat_ref,
                 wb_ref, wg_ref, wa_ref, wbt_ref):
    i = pl.program_id(0)
    wb_ref[...] = wbase_ref[...].astype(jnp.bfloat16).T

    @pl.when(i == 0)
    def _():
        wg_ref[...] = wgate_ref[...].astype(jnp.bfloat16).T
        wa_ref[...] = wa_in_ref[...].astype(jnp.bfloat16).T
        wbt_ref[...] = (SCALING * wbmat_ref[...]).astype(jnp.bfloat16).T


def _fused_kernel(x_ref, wg_ref, wa_ref, wb_ref, wbt_ref, out_ref):
    xb = x_ref[...].astype(jnp.bfloat16)
    logits = jnp.dot(xb, wg_ref[...], preferred_element_type=jnp.float32)
    ii = jax.lax.broadcasted_iota(jnp.int32, (BN, NE), 1)
    m1 = jnp.max(logits, axis=-1, keepdims=True)
    i1 = jnp.min(jnp.where(logits == m1, ii, NE), axis=-1, keepdims=True)
    l2 = jnp.where(ii == i1, -jnp.inf, logits)
    m2 = jnp.max(l2, axis=-1, keepdims=True)
    i2 = jnp.min(jnp.where(l2 == m2, ii, NE), axis=-1, keepdims=True)
    e = jnp.exp(m2 - m1)
    w1 = 1.0 / (1.0 + e)
    w2 = 1.0 - w1
    colmap = jax.lax.broadcasted_iota(jnp.int32, (BN, RMOE), 1) // R
    gex = jnp.where(colmap == i1, w1, 0.0) + jnp.where(colmap == i2, w2, 0.0)
    h = jnp.dot(xb, wa_ref[...], preferred_element_type=jnp.float32)
    hw = (h * gex).astype(jnp.bfloat16)
    out_ref[...] = jnp.dot(xb, wb_ref[...], preferred_element_type=jnp.float32)
    out_ref[...] += jnp.dot(hw, wbt_ref[...], preferred_element_type=jnp.float32)


@jax.jit
def kernel(x, W_base, W_gate, W_A, W_B):
    wb, wg, wa, wbt = pl.pallas_call(
        _prep_kernel,
        grid=(8,),
        in_specs=[
            pl.BlockSpec((WCH, D), lambda i: (i, 0)),
            pl.BlockSpec((NE, D), lambda i: (0, 0)),
            pl.BlockSpec((RMOE, D), lambda i: (0, 0)),
            pl.BlockSpec((OUT, RMOE), lambda i: (0, 0)),
        ],
        out_specs=[
            pl.BlockSpec((D, WCH), lambda i: (0, i)),
            pl.BlockSpec((D, NE), lambda i: (0, 0)),
            pl.BlockSpec((D, RMOE), lambda i: (0, 0)),
            pl.BlockSpec((RMOE, OUT), lambda i: (0, 0)),
        ],
        out_shape=[
            jax.ShapeDtypeStruct((D, OUT), jnp.bfloat16),
            jax.ShapeDtypeStruct((D, NE), jnp.bfloat16),
            jax.ShapeDtypeStruct((D, RMOE), jnp.bfloat16),
            jax.ShapeDtypeStruct((RMOE, OUT), jnp.bfloat16),
        ],
        compiler_params=pltpu.CompilerParams(
            vmem_limit_bytes=100 * 1024 * 1024,
        ),
    )(W_base, W_gate, W_A, W_B)

    return pl.pallas_call(
        _fused_kernel,
        grid=(N // BN,),
        in_specs=[
            pl.BlockSpec((BN, D), lambda i: (i, 0)),
            pl.BlockSpec((D, NE), lambda i: (0, 0)),
            pl.BlockSpec((D, RMOE), lambda i: (0, 0)),
            pl.BlockSpec((D, OUT), lambda i: (0, 0)),
            pl.BlockSpec((RMOE, OUT), lambda i: (0, 0)),
        ],
        out_specs=pl.BlockSpec((BN, OUT), lambda i: (i, 0)),
        out_shape=jax.ShapeDtypeStruct((N, OUT), jnp.float32),
        compiler_params=pltpu.CompilerParams(
            vmem_limit_bytes=100 * 1024 * 1024,
        ),
    )(x, wg, wa, wb, wbt)
